# baseline (speedup 1.0000x reference)
import functools

import jax
import jax.numpy as jnp
from jax import lax
from jax.experimental import pallas as pl
from jax.experimental.pallas import tpu as pltpu
from jax.experimental.pallas import tpu_sc as plsc

N = 10000
D = 128
E = 320000
H1 = 512
H2 = 256

NC = 2
NS = 16
NW = NC * NS
EPW = E // NW
CH = 125
NCHUNK = EPW // CH
NHALF = 2
ICH = NCHUNK // NHALF
WAVE = 8
DEG_PAD = 10240
DEG_STRIPE = DEG_PAD // NS
ACC_PAD = 10240
ROWS_PER_TILE = ACC_PAD // NS
RB = 80

_mesh = plsc.VectorSubcoreMesh(core_axis_name="c", subcore_axis_name="s")


@functools.partial(
    pl.kernel,
    mesh=_mesh,
    out_type=jax.ShapeDtypeStruct((NC, DEG_PAD), jnp.float32),
    scratch_types=[
        pltpu.VMEM((NCHUNK, CH), jnp.int32),
        pltpu.VMEM((CH,), jnp.float32),
        pltpu.VMEM((DEG_STRIPE,), jnp.float32),
        pltpu.VMEM_SHARED((DEG_PAD,), jnp.float32),
        pltpu.SemaphoreType.DMA,
    ],
)
def _deg_kernel(dst_hbm, ones_hbm, zeros_hbm, out_hbm,
                di_v, ones_v, zb_v, deg_sh, sem):
    cid = lax.axis_index("c")
    sid = lax.axis_index("s")
    wid = sid * NC + cid
    pltpu.sync_copy(dst_hbm.at[wid], di_v)
    pltpu.sync_copy(ones_hbm, ones_v)
    pltpu.sync_copy(zeros_hbm, zb_v)
    pltpu.sync_copy(zb_v, deg_sh.at[pl.ds(sid * DEG_STRIPE, DEG_STRIPE)])
    plsc.subcore_barrier()

    @pl.loop(0, NCHUNK, step=WAVE)
    def _waves(k):
        for w in range(WAVE):
            pltpu.async_copy(ones_v, deg_sh.at[di_v.at[k + w]], sem, add=True)
        for w in range(WAVE):
            pltpu.make_async_copy(
                ones_v, deg_sh.at[di_v.at[k]], sem).wait()

    plsc.subcore_barrier()
    pltpu.sync_copy(deg_sh.at[pl.ds(sid * DEG_STRIPE, DEG_STRIPE)], zb_v)
    pltpu.sync_copy(zb_v, out_hbm.at[cid, pl.ds(sid * DEG_STRIPE, DEG_STRIPE)])


@functools.partial(
    pl.kernel,
    mesh=_mesh,
    out_type=jax.ShapeDtypeStruct((NC, ACC_PAD, D), jnp.float32),
    scratch_types=[
        pltpu.VMEM((ICH, CH), jnp.int32),
        pltpu.VMEM((ICH, CH), jnp.int32),
        pltpu.VMEM((CH, D), jnp.float32),
        pltpu.VMEM((CH, D), jnp.float32),
        pltpu.VMEM_SHARED((ACC_PAD, D), jnp.float32),
        pltpu.SemaphoreType.DMA,
        pltpu.SemaphoreType.DMA,
        pltpu.SemaphoreType.DMA,
        pltpu.SemaphoreType.DMA,
    ],
)
def _accum_kernel(hs_hbm, src_hbm, dst_hbm, zrows_hbm, out_hbm,
                  si_v, di_v, rows0_v, rows1_v, acc_sh,
                  sem0, sem1, ssem0, ssem1):
    cid = lax.axis_index("c")
    sid = lax.axis_index("s")
    wid = sid * NC + cid
    row0 = sid * ROWS_PER_TILE
    stg = rows0_v.at[pl.ds(0, RB)]
    pltpu.sync_copy(zrows_hbm, stg)
    for s in range(ROWS_PER_TILE // RB):
        pltpu.sync_copy(stg, acc_sh.at[pl.ds(row0 + s * RB, RB)])
    plsc.subcore_barrier()

    rows = (rows0_v, rows1_v)
    sems = (sem0, sem1)
    ssems = (ssem0, ssem1)

    def _gather(j, b):
        pltpu.async_copy(hs_hbm.at[si_v.at[j]], rows[b], sems[b])

    def _wait(b):
        pltpu.make_async_copy(
            hs_hbm.at[si_v.at[0]], rows[b], sems[b]).wait()

    def _scatter(j, b):
        pltpu.async_copy(rows[b], acc_sh.at[di_v.at[j]], ssems[b], add=True)

    def _swait(b):
        pltpu.make_async_copy(
            rows[b], acc_sh.at[di_v.at[0]], ssems[b]).wait()

    for half in range(NHALF):
        pltpu.sync_copy(src_hbm.at[wid, half], si_v)
        pltpu.sync_copy(dst_hbm.at[wid, half], di_v)
        _gather(0, 0)
        _gather(1, 1)

        @pl.loop(0, ICH, step=2)
        def _chunks(k):
            _wait(0)
            _scatter(k, 0)
            _wait(1)
            _scatter(k + 1, 1)
            _swait(0)
            _gather(jnp.minimum(k + 2, ICH - 1), 0)
            _swait(1)
            _gather(jnp.minimum(k + 3, ICH - 1), 1)

        _wait(0)
        _wait(1)

    plsc.subcore_barrier()
    for s in range(ROWS_PER_TILE // RB):
        pltpu.sync_copy(acc_sh.at[pl.ds(row0 + s * RB, RB)], stg)
        pltpu.sync_copy(stg, out_hbm.at[cid, pl.ds(row0 + s * RB, RB)])


_HS_RB = 2000
_HS_GRID = N // _HS_RB


def _hs_body(x_ref, w_ref, degp_ref, hs_ref):
    deg = degp_ref[:, 0] + degp_ref[:, 1] + 1.0
    dinv = lax.rsqrt(deg)
    h = jnp.dot(x_ref[...], w_ref[...], preferred_element_type=jnp.float32)
    hs_ref[...] = h * dinv[:, None]


def _hs_stage(x, w_gcn, degp):
    return pl.pallas_call(
        _hs_body,
        grid=(_HS_GRID,),
        in_specs=[
            pl.BlockSpec((_HS_RB, D), lambda i: (i, 0)),
            pl.BlockSpec((D, D), lambda i: (0, 0)),
            pl.BlockSpec((_HS_RB, NC), lambda i: (i, 0)),
        ],
        out_specs=pl.BlockSpec((_HS_RB, D), lambda i: (i, 0)),
        out_shape=jax.ShapeDtypeStruct((N, D), jnp.float32),
    )(x, w_gcn, degp)


def _epi_body(acc_ref, hs_ref, x_ref, degp_ref, bg_ref,
              w1_ref, b1_ref, w2_ref, b2_ref, w3_ref, b3_ref,
              out_ref, tacc):
    i = pl.program_id(0)

    @pl.when(i == 0)
    def _():
        tacc[...] = jnp.zeros((1, D), jnp.float32)

    sl = pl.ds(i * _HS_RB, _HS_RB)
    deg = degp_ref[:, 0] + degp_ref[:, 1] + 1.0
    dinv = lax.rsqrt(deg)
    acc = acc_ref[0, sl, :] + acc_ref[1, sl, :] + hs_ref[...]
    conv = dinv[:, None] * acc + bg_ref[...]
    h2 = jnp.maximum(conv, 0.0) + x_ref[...]
    tacc[...] += jnp.sum(h2, axis=0, keepdims=True)

    @pl.when(i == _HS_GRID - 1)
    def _():
        t = tacc[...]
        z1 = jnp.maximum(
            jnp.dot(t, w1_ref[...], preferred_element_type=jnp.float32)
            + b1_ref[...], 0.0)
        z2 = jnp.maximum(
            jnp.dot(z1, w2_ref[...], preferred_element_type=jnp.float32)
            + b2_ref[...], 0.0)
        out_ref[...] = (jnp.dot(z2, w3_ref[...],
                                preferred_element_type=jnp.float32)
                        + b3_ref[...])


def _epilogue(acc2, hs, x, degp, b_gcn, w1, b1, w2, b2, w3, b3):
    return pl.pallas_call(
        _epi_body,
        grid=(_HS_GRID,),
        in_specs=[
            pl.BlockSpec((NC, ACC_PAD, D), lambda i: (0, 0, 0)),
            pl.BlockSpec((_HS_RB, D), lambda i: (i, 0)),
            pl.BlockSpec((_HS_RB, D), lambda i: (i, 0)),
            pl.BlockSpec((_HS_RB, NC), lambda i: (i, 0)),
            pl.BlockSpec((1, D), lambda i: (0, 0)),
            pl.BlockSpec((D, H1), lambda i: (0, 0)),
            pl.BlockSpec((1, H1), lambda i: (0, 0)),
            pl.BlockSpec((H1, H2), lambda i: (0, 0)),
            pl.BlockSpec((1, H2), lambda i: (0, 0)),
            pl.BlockSpec((H2, 1), lambda i: (0, 0)),
            pl.BlockSpec((1, 1), lambda i: (0, 0)),
        ],
        out_specs=pl.BlockSpec((1, 1), lambda i: (0, 0)),
        out_shape=jax.ShapeDtypeStruct((1, 1), jnp.float32),
        scratch_shapes=[pltpu.VMEM((1, D), jnp.float32)],
    )(acc2, hs, x, degp, b_gcn, w1, b1, w2, b2, w3, b3)


def kernel(x, edge_index, W_gcn, b_gcn, W1, b1, W2, b2, W3, b3):
    src4 = edge_index[0].reshape(NW, NHALF, ICH, CH)
    dst4 = edge_index[1].reshape(NW, NHALF, ICH, CH)
    dst3 = edge_index[1].reshape(NW, NCHUNK, CH)
    ones_ch = jnp.ones((CH,), jnp.float32)
    zeros_deg = jnp.zeros((DEG_STRIPE,), jnp.float32)
    zeros_rows = jnp.zeros((RB, D), jnp.float32)

    degp = _deg_kernel(dst3, ones_ch, zeros_deg)
    degp = degp[:, :N].T
    hs = _hs_stage(x, W_gcn, degp)
    acc2 = _accum_kernel(hs, src4, dst4, zeros_rows)
    out = _epilogue(acc2, hs, x, degp,
                    b_gcn.reshape(1, D),
                    W1, b1.reshape(1, H1),
                    W2, b2.reshape(1, H2),
                    W3, b3.reshape(1, 1))
    return out.reshape(1)


# speedup vs baseline: 1.2097x; 1.2097x over previous
import functools

import jax
import jax.numpy as jnp
from jax import lax
from jax.experimental import pallas as pl
from jax.experimental.pallas import tpu as pltpu
from jax.experimental.pallas import tpu_sc as plsc

N = 10000
D = 128
E = 320000
H1 = 512
H2 = 256

NC = 2
NS = 16
NW = NC * NS
EPW = E // NW
CH = 125
NCHUNK = EPW // CH
NHALF = 2
ICH = NCHUNK // NHALF
WAVE = 8
DEG_PAD = 10240
DEG_STRIPE = DEG_PAD // NS
DEG_NCHUNK = E // NS // CH
DEG_HALF = DEG_PAD // NC // NS
ACC_PAD = 10240
ROWS_PER_TILE = ACC_PAD // NS
RB = 80

_mesh = plsc.VectorSubcoreMesh(core_axis_name="c", subcore_axis_name="s")


@functools.partial(
    pl.kernel,
    mesh=_mesh,
    out_type=jax.ShapeDtypeStruct((NC, DEG_PAD), jnp.float32),
    scratch_types=[
        pltpu.VMEM((NCHUNK, CH), jnp.int32),
        pltpu.VMEM((CH,), jnp.float32),
        pltpu.VMEM((DEG_STRIPE,), jnp.float32),
        pltpu.VMEM_SHARED((DEG_PAD,), jnp.float32),
        pltpu.SemaphoreType.DMA,
    ],
)
def _deg_kernel(dst_hbm, ones_hbm, zeros_hbm, out_hbm,
                di_v, ones_v, zb_v, deg_sh, sem):
    cid = lax.axis_index("c")
    sid = lax.axis_index("s")
    wid = sid * NC + cid
    pltpu.sync_copy(dst_hbm.at[wid], di_v)
    pltpu.sync_copy(ones_hbm, ones_v)
    pltpu.sync_copy(zeros_hbm, zb_v)
    pltpu.sync_copy(zb_v, deg_sh.at[pl.ds(sid * DEG_STRIPE, DEG_STRIPE)])
    plsc.subcore_barrier()

    @pl.loop(0, NCHUNK, step=WAVE)
    def _waves(k):
        for w in range(WAVE):
            pltpu.async_copy(ones_v, deg_sh.at[di_v.at[k + w]], sem, add=True)
        for w in range(WAVE):
            pltpu.make_async_copy(
                ones_v, deg_sh.at[di_v.at[k]], sem).wait()

    plsc.subcore_barrier()
    pltpu.sync_copy(deg_sh.at[pl.ds(sid * DEG_STRIPE, DEG_STRIPE)], zb_v)
    pltpu.sync_copy(zb_v, out_hbm.at[cid, pl.ds(sid * DEG_STRIPE, DEG_STRIPE)])


@functools.partial(
    pl.kernel,
    mesh=_mesh,
    out_type=jax.ShapeDtypeStruct((NC, ACC_PAD, D), jnp.float32),
    scratch_types=[
        pltpu.VMEM((ICH, CH), jnp.int32),
        pltpu.VMEM((ICH, CH), jnp.int32),
        pltpu.VMEM((CH, D), jnp.float32),
        pltpu.VMEM((CH, D), jnp.float32),
        pltpu.VMEM_SHARED((ACC_PAD, D), jnp.float32),
        pltpu.SemaphoreType.DMA,
        pltpu.SemaphoreType.DMA,
    ],
)
def _accum_kernel(hs_hbm, src_hbm, dst_hbm, zrows_hbm, out_hbm,
                  si_v, di_v, rows0_v, rows1_v, acc_sh, sem0, sem1):
    cid = lax.axis_index("c")
    sid = lax.axis_index("s")
    wid = sid * NC + cid
    row0 = sid * ROWS_PER_TILE
    stg = rows0_v.at[pl.ds(0, RB)]
    pltpu.sync_copy(zrows_hbm, stg)
    for s in range(ROWS_PER_TILE // RB):
        pltpu.sync_copy(stg, acc_sh.at[pl.ds(row0 + s * RB, RB)])
    plsc.subcore_barrier()

    rows = (rows0_v, rows1_v)
    sems = (sem0, sem1)

    def _gather(j, b):
        pltpu.async_copy(hs_hbm.at[si_v.at[j]], rows[b], sems[b])

    def _wait(b):
        pltpu.make_async_copy(
            hs_hbm.at[si_v.at[0]], rows[b], sems[b]).wait()

    for half in range(NHALF):
        pltpu.sync_copy(src_hbm.at[wid, half], si_v)
        pltpu.sync_copy(dst_hbm.at[wid, half], di_v)
        _gather(0, 0)
        _gather(1, 1)

        @pl.loop(0, ICH, step=2)
        def _chunks(k):
            for b in range(2):
                _wait(b)
                pltpu.sync_copy(rows[b], acc_sh.at[di_v.at[k + b]], add=True)
                _gather(jnp.minimum(k + 2 + b, ICH - 1), b)

        _wait(0)
        _wait(1)

    plsc.subcore_barrier()
    for s in range(ROWS_PER_TILE // RB):
        pltpu.sync_copy(acc_sh.at[pl.ds(row0 + s * RB, RB)], stg)
        pltpu.sync_copy(stg, out_hbm.at[cid, pl.ds(row0 + s * RB, RB)])


_HS_RB = 2000
_HS_GRID = N // _HS_RB


def _hs_body(x_ref, w_ref, degp_ref, hs_ref):
    deg = degp_ref[:, 0] + degp_ref[:, 1] + 1.0
    dinv = lax.rsqrt(deg)
    h = jnp.dot(x_ref[...], w_ref[...], preferred_element_type=jnp.float32)
    hs_ref[...] = h * dinv[:, None]


def _hs_stage(x, w_gcn, degp):
    return pl.pallas_call(
        _hs_body,
        grid=(_HS_GRID,),
        in_specs=[
            pl.BlockSpec((_HS_RB, D), lambda i: (i, 0)),
            pl.BlockSpec((D, D), lambda i: (0, 0)),
            pl.BlockSpec((_HS_RB, NC), lambda i: (i, 0)),
        ],
        out_specs=pl.BlockSpec((_HS_RB, D), lambda i: (i, 0)),
        out_shape=jax.ShapeDtypeStruct((N, D), jnp.float32),
    )(x, w_gcn, degp)


def _epi_body(acc_ref, hs_ref, x_ref, degp_ref, bg_ref,
              w1_ref, b1_ref, w2_ref, b2_ref, w3_ref, b3_ref,
              out_ref, tacc):
    i = pl.program_id(0)

    @pl.when(i == 0)
    def _():
        tacc[...] = jnp.zeros((1, D), jnp.float32)

    deg = degp_ref[:, 0] + degp_ref[:, 1] + 1.0
    dinv = lax.rsqrt(deg)
    acc = acc_ref[0] + acc_ref[1] + hs_ref[...]
    conv = dinv[:, None] * acc + bg_ref[...]
    h2 = jnp.maximum(conv, 0.0) + x_ref[...]
    tacc[...] += jnp.sum(h2, axis=0, keepdims=True)

    @pl.when(i == _HS_GRID - 1)
    def _():
        t = tacc[...]
        z1 = jnp.maximum(
            jnp.dot(t, w1_ref[...], preferred_element_type=jnp.float32)
            + b1_ref[...], 0.0)
        z2 = jnp.maximum(
            jnp.dot(z1, w2_ref[...], preferred_element_type=jnp.float32)
            + b2_ref[...], 0.0)
        out_ref[...] = (jnp.dot(z2, w3_ref[...],
                                preferred_element_type=jnp.float32)
                        + b3_ref[...])


def _epilogue(acc2, hs, x, degp, b_gcn, w1, b1, w2, b2, w3, b3):
    return pl.pallas_call(
        _epi_body,
        grid=(_HS_GRID,),
        in_specs=[
            pl.BlockSpec((NC, _HS_RB, D), lambda i: (0, i, 0)),
            pl.BlockSpec((_HS_RB, D), lambda i: (i, 0)),
            pl.BlockSpec((_HS_RB, D), lambda i: (i, 0)),
            pl.BlockSpec((_HS_RB, NC), lambda i: (i, 0)),
            pl.BlockSpec((1, D), lambda i: (0, 0)),
            pl.BlockSpec((D, H1), lambda i: (0, 0)),
            pl.BlockSpec((1, H1), lambda i: (0, 0)),
            pl.BlockSpec((H1, H2), lambda i: (0, 0)),
            pl.BlockSpec((1, H2), lambda i: (0, 0)),
            pl.BlockSpec((H2, 1), lambda i: (0, 0)),
            pl.BlockSpec((1, 1), lambda i: (0, 0)),
        ],
        out_specs=pl.BlockSpec((1, 1), lambda i: (0, 0)),
        out_shape=jax.ShapeDtypeStruct((1, 1), jnp.float32),
        scratch_shapes=[pltpu.VMEM((1, D), jnp.float32)],
    )(acc2, hs, x, degp, b_gcn, w1, b1, w2, b2, w3, b3)


def kernel(x, edge_index, W_gcn, b_gcn, W1, b1, W2, b2, W3, b3):
    src4 = edge_index[0].reshape(NW, NHALF, ICH, CH)
    dst4 = edge_index[1].reshape(NW, NHALF, ICH, CH)
    dst3 = edge_index[1].reshape(NW, NCHUNK, CH)
    ones_ch = jnp.ones((CH,), jnp.float32)
    zeros_deg = jnp.zeros((DEG_STRIPE,), jnp.float32)
    zeros_rows = jnp.zeros((RB, D), jnp.float32)

    degp = _deg_kernel(dst3, ones_ch, zeros_deg)
    degp = degp[:, :N].T
    hs = _hs_stage(x, W_gcn, degp)
    acc2 = _accum_kernel(hs, src4, dst4, zeros_rows)
    out = _epilogue(acc2, hs, x, degp,
                    b_gcn.reshape(1, D),
                    W1, b1.reshape(1, H1),
                    W2, b2.reshape(1, H2),
                    W3, b3.reshape(1, 1))
    return out.reshape(1)


# speedup vs baseline: 1.2772x; 1.0558x over previous
import functools

import jax
import jax.numpy as jnp
from jax import lax
from jax.experimental import pallas as pl
from jax.experimental.pallas import tpu as pltpu
from jax.experimental.pallas import tpu_sc as plsc

N = 10000
D = 128
E = 320000
H1 = 512
H2 = 256

NC = 2
NS = 16
NW = NC * NS
EPW = E // NW
CH = 125
NCHUNK = EPW // CH
NHALF = 2
ICH = NCHUNK // NHALF
WAVE = 8
DEG_PAD = 10240
DEG_STRIPE = DEG_PAD // NS
DEG_NCHUNK = E // NS // CH
DEG_HALF = DEG_PAD // NC // NS
ACC_PAD = 10240
ROWS_PER_TILE = ACC_PAD // NS
RB = 80

_mesh = plsc.VectorSubcoreMesh(core_axis_name="c", subcore_axis_name="s")


@functools.partial(
    pl.kernel,
    mesh=_mesh,
    out_type=jax.ShapeDtypeStruct((NC, DEG_PAD), jnp.float32),
    scratch_types=[
        pltpu.VMEM((NCHUNK, CH), jnp.int32),
        pltpu.VMEM((CH,), jnp.float32),
        pltpu.VMEM((DEG_STRIPE,), jnp.float32),
        pltpu.VMEM_SHARED((DEG_PAD,), jnp.float32),
        pltpu.SemaphoreType.DMA,
    ],
)
def _deg_kernel(ei_hbm, ones_hbm, zeros_hbm, out_hbm,
                di_v, ones_v, zb_v, deg_sh, sem):
    cid = lax.axis_index("c")
    sid = lax.axis_index("s")
    wid = sid * NC + cid
    pltpu.sync_copy(ei_hbm.at[NW + wid], di_v)
    pltpu.sync_copy(ones_hbm, ones_v)
    pltpu.sync_copy(zeros_hbm, zb_v)
    pltpu.sync_copy(zb_v, deg_sh.at[pl.ds(sid * DEG_STRIPE, DEG_STRIPE)])
    plsc.subcore_barrier()

    @pl.loop(0, NCHUNK, step=WAVE)
    def _waves(k):
        for w in range(WAVE):
            pltpu.async_copy(ones_v, deg_sh.at[di_v.at[k + w]], sem, add=True)
        for w in range(WAVE):
            pltpu.make_async_copy(
                ones_v, deg_sh.at[di_v.at[k]], sem).wait()

    plsc.subcore_barrier()
    pltpu.sync_copy(deg_sh.at[pl.ds(sid * DEG_STRIPE, DEG_STRIPE)], zb_v)
    pltpu.sync_copy(zb_v, out_hbm.at[cid, pl.ds(sid * DEG_STRIPE, DEG_STRIPE)])


@functools.partial(
    pl.kernel,
    mesh=_mesh,
    out_type=jax.ShapeDtypeStruct((NC, ACC_PAD, D), jnp.float32),
    scratch_types=[
        pltpu.VMEM((ICH, CH), jnp.int32),
        pltpu.VMEM((ICH, CH), jnp.int32),
        pltpu.VMEM((CH, D), jnp.float32),
        pltpu.VMEM((CH, D), jnp.float32),
        pltpu.VMEM_SHARED((ACC_PAD, D), jnp.float32),
        pltpu.SemaphoreType.DMA,
        pltpu.SemaphoreType.DMA,
    ],
)
def _accum_kernel(hs_hbm, ei_hbm, zrows_hbm, out_hbm,
                  si_v, di_v, rows0_v, rows1_v, acc_sh, sem0, sem1):
    cid = lax.axis_index("c")
    sid = lax.axis_index("s")
    wid = sid * NC + cid
    row0 = sid * ROWS_PER_TILE
    stg = rows0_v.at[pl.ds(0, RB)]
    pltpu.sync_copy(zrows_hbm, stg)
    for s in range(ROWS_PER_TILE // RB):
        pltpu.sync_copy(stg, acc_sh.at[pl.ds(row0 + s * RB, RB)])
    plsc.subcore_barrier()

    rows = (rows0_v, rows1_v)
    sems = (sem0, sem1)

    def _gather(j, b):
        pltpu.async_copy(hs_hbm.at[si_v.at[j]], rows[b], sems[b])

    def _wait(b):
        pltpu.make_async_copy(
            hs_hbm.at[si_v.at[0]], rows[b], sems[b]).wait()

    for half in range(NHALF):
        pltpu.sync_copy(ei_hbm.at[wid, half], si_v)
        pltpu.sync_copy(ei_hbm.at[NW + wid, half], di_v)
        _gather(0, 0)
        _gather(1, 1)

        @pl.loop(0, ICH, step=2)
        def _chunks(k):
            for b in range(2):
                _wait(b)
                pltpu.sync_copy(rows[b], acc_sh.at[di_v.at[k + b]], add=True)
                _gather(jnp.minimum(k + 2 + b, ICH - 1), b)

        _wait(0)
        _wait(1)

    plsc.subcore_barrier()
    for s in range(ROWS_PER_TILE // RB):
        pltpu.sync_copy(acc_sh.at[pl.ds(row0 + s * RB, RB)], stg)
        pltpu.sync_copy(stg, out_hbm.at[cid, pl.ds(row0 + s * RB, RB)])


_HS_RB = 2000
_HS_GRID = N // _HS_RB


def _hs_body(x_ref, w_ref, degp_ref, hs_ref):
    deg = degp_ref[:, 0] + degp_ref[:, 1] + 1.0
    dinv = lax.rsqrt(deg)
    h = jnp.dot(x_ref[...], w_ref[...], preferred_element_type=jnp.float32)
    hs_ref[...] = h * dinv[:, None]


def _hs_stage(x, w_gcn, degp):
    return pl.pallas_call(
        _hs_body,
        grid=(_HS_GRID,),
        in_specs=[
            pl.BlockSpec((_HS_RB, D), lambda i: (i, 0)),
            pl.BlockSpec((D, D), lambda i: (0, 0)),
            pl.BlockSpec((_HS_RB, NC), lambda i: (i, 0)),
        ],
        out_specs=pl.BlockSpec((_HS_RB, D), lambda i: (i, 0)),
        out_shape=jax.ShapeDtypeStruct((N, D), jnp.float32),
    )(x, w_gcn, degp)


def _epi_body(acc_ref, hs_ref, x_ref, degp_ref, bg_ref,
              w1_ref, b1_ref, w2_ref, b2_ref, w3_ref, b3_ref,
              out_ref, tacc):
    i = pl.program_id(0)

    @pl.when(i == 0)
    def _():
        tacc[...] = jnp.zeros((1, D), jnp.float32)

    deg = degp_ref[:, 0] + degp_ref[:, 1] + 1.0
    dinv = lax.rsqrt(deg)
    acc = acc_ref[0] + acc_ref[1] + hs_ref[...]
    conv = dinv[:, None] * acc + bg_ref[...]
    h2 = jnp.maximum(conv, 0.0) + x_ref[...]
    tacc[...] += jnp.sum(h2, axis=0, keepdims=True)

    @pl.when(i == _HS_GRID - 1)
    def _():
        t = tacc[...]
        z1 = jnp.maximum(
            jnp.dot(t, w1_ref[...], preferred_element_type=jnp.float32)
            + b1_ref[...], 0.0)
        z2 = jnp.maximum(
            jnp.dot(z1, w2_ref[...], preferred_element_type=jnp.float32)
            + b2_ref[...], 0.0)
        out_ref[...] = (jnp.dot(z2, w3_ref[...],
                                preferred_element_type=jnp.float32)
                        + b3_ref[...])


def _epilogue(acc2, hs, x, degp, b_gcn, w1, b1, w2, b2, w3, b3):
    return pl.pallas_call(
        _epi_body,
        grid=(_HS_GRID,),
        in_specs=[
            pl.BlockSpec((NC, _HS_RB, D), lambda i: (0, i, 0)),
            pl.BlockSpec((_HS_RB, D), lambda i: (i, 0)),
            pl.BlockSpec((_HS_RB, D), lambda i: (i, 0)),
            pl.BlockSpec((_HS_RB, NC), lambda i: (i, 0)),
            pl.BlockSpec((1, D), lambda i: (0, 0)),
            pl.BlockSpec((D, H1), lambda i: (0, 0)),
            pl.BlockSpec((1, H1), lambda i: (0, 0)),
            pl.BlockSpec((H1, H2), lambda i: (0, 0)),
            pl.BlockSpec((1, H2), lambda i: (0, 0)),
            pl.BlockSpec((H2, 1), lambda i: (0, 0)),
            pl.BlockSpec((1, 1), lambda i: (0, 0)),
        ],
        out_specs=pl.BlockSpec((1, 1), lambda i: (0, 0)),
        out_shape=jax.ShapeDtypeStruct((1, 1), jnp.float32),
        scratch_shapes=[pltpu.VMEM((1, D), jnp.float32)],
    )(acc2, hs, x, degp, b_gcn, w1, b1, w2, b2, w3, b3)


def kernel(x, edge_index, W_gcn, b_gcn, W1, b1, W2, b2, W3, b3):
    ei4 = edge_index.reshape(2 * NW, NHALF, ICH, CH)
    ei3 = edge_index.reshape(2 * NW, NCHUNK, CH)
    ones_ch = jnp.ones((CH,), jnp.float32)
    zeros_deg = jnp.zeros((DEG_STRIPE,), jnp.float32)
    zeros_rows = jnp.zeros((RB, D), jnp.float32)

    degp = _deg_kernel(ei3, ones_ch, zeros_deg)
    degp = degp[:, :N].T
    hs = _hs_stage(x, W_gcn, degp)
    acc2 = _accum_kernel(hs, ei4, zeros_rows)
    out = _epilogue(acc2, hs, x, degp,
                    b_gcn.reshape(1, D),
                    W1, b1.reshape(1, H1),
                    W2, b2.reshape(1, H2),
                    W3, b3.reshape(1, 1))
    return out.reshape(1)
